# Initial kernel scaffold; baseline (speedup 1.0000x reference)
import jax
import jax.numpy as jnp
from jax.experimental import pallas as pl


def kernel(point_faces, index_face, W1, gamma1, beta1, W2, gamma2, beta2):
    raise NotImplementedError("write your pallas kernel here")


# speedup vs baseline: 3.3276x; 3.3276x over previous
import functools

import jax
import jax.numpy as jnp
from jax import lax
from jax.experimental import pallas as pl
from jax.experimental.pallas import tpu as pltpu

NUM_GROUP = 2000
GROUP_SIZE = 32
ROW_BLK = 256
CTR_BLK = 256
BIGF = 1e30


def _adj_body(idxc_ref, idxr_ref, a_ref, deg_ref):
    ic = idxc_ref[0]
    ir = idxr_ref[0]
    acc = None
    for a in range(3):
        col = ic[:, a:a + 1]
        for b in range(3):
            row = ir[b:b + 1, :]
            e = col == row
            acc = e if acc is None else (acc | e)
    af = jnp.where(acc, 1.0, 0.0).astype(jnp.float32)
    a_ref[0] = af
    deg_ref[0] = jnp.sum(af, axis=1, keepdims=True)


def _adjacency(index_face, idx_t):
    bsz, n, _ = index_face.shape
    nb = n // ROW_BLK
    return pl.pallas_call(
        _adj_body,
        grid=(bsz, nb),
        in_specs=[
            pl.BlockSpec((1, ROW_BLK, 3), lambda b, r: (b, r, 0)),
            pl.BlockSpec((1, 3, n), lambda b, r: (b, 0, 0)),
        ],
        out_specs=[
            pl.BlockSpec((1, ROW_BLK, n), lambda b, r: (b, r, 0)),
            pl.BlockSpec((1, ROW_BLK, 1), lambda b, r: (b, r, 0)),
        ],
        out_shape=[
            jax.ShapeDtypeStruct((bsz, n, n), jnp.float32),
            jax.ShapeDtypeStruct((bsz, n, 1), jnp.float32),
        ],
    )(index_face, idx_t)


def _convbn_body(x_ref, wt_ref, g_ref, b_ref, o_ref):
    bsz, n, _ = x_ref.shape
    wt = wt_ref[...]
    hs = [jnp.dot(x_ref[i], wt, preferred_element_type=jnp.float32, precision=lax.Precision.HIGHEST)
          for i in range(bsz)]
    tot = float(bsz * n)
    mean = sum(jnp.sum(h, axis=0, keepdims=True) for h in hs) / tot
    ds = [h - mean for h in hs]
    var = sum(jnp.sum(d * d, axis=0, keepdims=True) for d in ds) / tot
    inv = 1.0 / jnp.sqrt(var + 1e-5)
    g = g_ref[...]
    beta = b_ref[...]
    for i in range(bsz):
        y = ds[i] * inv * g + beta
        o_ref[i] = jnp.where(y >= 0, y, 0.2 * y)


def _convbn(x, wt, gamma, beta):
    bsz, n, _ = x.shape
    cout = wt.shape[1]
    return pl.pallas_call(
        _convbn_body,
        out_shape=jax.ShapeDtypeStruct((bsz, n, cout), jnp.float32),
    )(x, wt, gamma, beta)


def _spmm_body(a_ref, deg_ref, h_ref, o_ref):
    h = h_ref[0]
    h_hi = h.astype(jnp.bfloat16).astype(jnp.float32)
    h_lo = h - h_hi
    a = a_ref[0]
    y = (jnp.dot(a, h_hi, preferred_element_type=jnp.float32)
         + jnp.dot(a, h_lo, preferred_element_type=jnp.float32))
    o_ref[0] = y / deg_ref[0]


def _spmm(a, deg, h):
    bsz, n, _ = a.shape
    c = h.shape[2]
    nb = n // ROW_BLK
    return pl.pallas_call(
        _spmm_body,
        grid=(bsz, nb),
        in_specs=[
            pl.BlockSpec((1, ROW_BLK, n), lambda b, r: (b, r, 0)),
            pl.BlockSpec((1, ROW_BLK, 1), lambda b, r: (b, r, 0)),
            pl.BlockSpec((1, n, c), lambda b, r: (b, 0, 0)),
        ],
        out_specs=pl.BlockSpec((1, ROW_BLK, c), lambda b, r: (b, r, 0)),
        out_shape=jax.ShapeDtypeStruct((bsz, n, c), jnp.float32),
    )(a, deg, h)


def _fpsknn_body(pf_ref, ptst_ref, xyz_ref, ctr_ref, nbh_ref, ctrs_ref):
    n = pf_ref.shape[1]
    nsub = n // 128
    xs = xyz_ref[0, 0]
    ys = xyz_ref[0, 1]
    zs = xyz_ref[0, 2]
    iota2 = (lax.broadcasted_iota(jnp.int32, (nsub, 128), 0) * 128
             + lax.broadcasted_iota(jnp.int32, (nsub, 128), 1))
    ctrs_ref[...] = jnp.zeros((n, 16), jnp.float32)

    def fps_step(t, carry):
        distv, f = carry
        ctrs_ref[pl.ds(t, 1), 0:12] = pf_ref[0, pl.ds(f, 1), 0:12]
        row = pf_ref[0, pl.ds(f, 1), 9:12]
        dx = xs - row[0, 0]
        dy = ys - row[0, 1]
        dz = zs - row[0, 2]
        distv = jnp.minimum(distv, dx * dx + dy * dy + dz * dz)
        m = jnp.max(distv)
        sel = jnp.where(distv == m, iota2, 2 * n)
        return distv, jnp.min(sel)

    init = (jnp.full((nsub, 128), 1e10, jnp.float32), jnp.int32(0))
    lax.fori_loop(0, NUM_GROUP, fps_step, init)

    pts12 = pf_ref[0, :, :12]
    ptst12 = ptst_ref[0]
    pts_hi = pts12.astype(jnp.bfloat16).astype(jnp.float32)
    pts_lo = pts12 - pts_hi
    ctr_ref[0] = ctrs_ref[0:NUM_GROUP, 0:12]

    lane_iota = lax.broadcasted_iota(jnp.int32, (CTR_BLK, n), 1)
    ncb = n // CTR_BLK
    for cb in range(ncb):
        c0 = cb * CTR_BLK
        dblk = None
        for d in range(12):
            dif = ctrs_ref[pl.ds(c0, CTR_BLK), d:d + 1] - ptst12[d:d + 1, :]
            dblk = dif * dif if dblk is None else dblk + dif * dif
        cblk = ctrs_ref[pl.ds(c0, CTR_BLK), 0:12]
        for k in range(GROUP_SIZE):
            m = jnp.min(dblk, axis=1, keepdims=True)
            sel = jnp.min(jnp.where(dblk == m, lane_iota, 2 * n),
                          axis=1, keepdims=True)
            oh = lane_iota == sel
            ohf = jnp.where(oh, 1.0, 0.0)
            dblk = jnp.where(oh, BIGF, dblk)
            nbk = (jnp.dot(ohf, pts_hi, preferred_element_type=jnp.float32)
                   + jnp.dot(ohf, pts_lo, preferred_element_type=jnp.float32))
            nbh_ref[0, pl.ds(c0, CTR_BLK), k * 12:(k + 1) * 12] = nbk - cblk


def _fpsknn(pf, ptst, xyzp):
    bsz, n, _ = pf.shape
    return pl.pallas_call(
        _fpsknn_body,
        grid=(bsz,),
        in_specs=[
            pl.BlockSpec((1, n, 24), lambda b: (b, 0, 0)),
            pl.BlockSpec((1, 12, n), lambda b: (b, 0, 0)),
            pl.BlockSpec((1, 3, n // 128, 128), lambda b: (b, 0, 0, 0)),
        ],
        out_specs=[
            pl.BlockSpec((1, NUM_GROUP, 12), lambda b: (b, 0, 0)),
            pl.BlockSpec((1, n, 12 * GROUP_SIZE), lambda b: (b, 0, 0)),
        ],
        out_shape=[
            jax.ShapeDtypeStruct((bsz, NUM_GROUP, 12), jnp.float32),
            jax.ShapeDtypeStruct((bsz, n, 12 * GROUP_SIZE), jnp.float32),
        ],
        scratch_shapes=[
            pltpu.VMEM((n, 16), jnp.float32),
        ],
    )(pf, ptst, xyzp)


def kernel(point_faces, index_face, W1, gamma1, beta1, W2, gamma2, beta2):
    bsz, n, _ = point_faces.shape
    idx_t = jnp.transpose(index_face, (0, 2, 1))
    ptst = jnp.transpose(point_faces[:, :, :12], (0, 2, 1))
    xyzp = ptst[:, 9:12, :].reshape(bsz, 3, n // 128, 128)
    feats1 = point_faces[:, :, 12:24]

    a, deg = _adjacency(index_face, idx_t)
    h1 = _convbn(feats1, W1.T, gamma1.reshape(1, -1), beta1.reshape(1, -1))
    y1 = _spmm(a, deg, h1)
    h2 = _convbn(y1, W2.T, gamma2.reshape(1, -1), beta2.reshape(1, -1))
    y2 = _spmm(a, deg, h2)
    nor2_1 = jnp.transpose(y2, (0, 2, 1))

    center, nbh = _fpsknn(point_faces, ptst, xyzp)
    neighborhood = nbh.reshape(bsz, n, GROUP_SIZE, 12)[:, :NUM_GROUP]
    return (nor2_1, neighborhood, center)
